# Initial kernel scaffold; baseline (speedup 1.0000x reference)
import jax
import jax.numpy as jnp
from jax.experimental import pallas as pl


def kernel(x, edge_index, edge_weight, W_lin, b_lin, W_u, U_u, b_u, W_r, U_r, b_r, W_h, U_h, b_h):
    raise NotImplementedError("write your pallas kernel here")


# speedup vs baseline: 6.5278x; 6.5278x over previous
import functools

import jax
import jax.numpy as jnp
from jax import lax
from jax.experimental import pallas as pl
from jax.experimental.pallas import tpu as pltpu
from jax.experimental.pallas import tpu_sc as plsc


def _tc_body(x_ref, wl_ref, wu_ref, uu_ref, bu_ref, wr_ref, ur_ref, br_ref,
             wh_ref, uh_ref, bh_ref, lo_ref, hi_ref, wp_ref):
    @pl.when(pl.program_id(0) == 0)
    def _():
        w = wl_ref[...]

        def mm(a, b):
            return jnp.dot(a, b, preferred_element_type=jnp.float32)

        def sig(z):
            return 1.0 / (1.0 + jnp.exp(-z))

        upd = sig(mm(wu_ref[...], w) + mm(uu_ref[...], w) + bu_ref[...])
        rst = sig(mm(wr_ref[...], w) + mm(ur_ref[...], w) + br_ref[...])
        hc = jnp.tanh(mm(wh_ref[...], w) + mm(uh_ref[...], rst * w)
                      + bh_ref[...])
        wp_ref[...] = (1.0 - upd) * w + upd * hc

    y = lax.dot_general(x_ref[...], wp_ref[...],
                        dimension_numbers=(((1,), (1,)), ((), ())),
                        preferred_element_type=jnp.float32)
    lo_ref[...] = y[:, :128]
    hi_ref[...] = y[:, 128:]


def _project(x, w_lin, w_u, u_u, b_u, w_r, u_r, b_r, w_h, u_h, b_h):
    n, d = x.shape
    blk = 1000
    grid = n // blk
    wspec = pl.BlockSpec((d, d), lambda i: (0, 0))
    return pl.pallas_call(
        _tc_body,
        grid=(grid,),
        in_specs=[pl.BlockSpec((blk, d), lambda i: (i, 0))] + [wspec] * 10,
        out_specs=[pl.BlockSpec((blk, d // 2), lambda i: (i, 0))] * 2,
        out_shape=[jax.ShapeDtypeStruct((n, d // 2), jnp.float32)] * 2,
        scratch_shapes=[pltpu.VMEM((d, d), jnp.float32)],
    )(x, w_lin, w_u, u_u, b_u, w_r, u_r, b_r, w_h, u_h, b_h)


_CH = 128


def _fast_rsqrt(v):
    i = lax.bitcast_convert_type(v, jnp.int32)
    y = lax.bitcast_convert_type(jnp.int32(0x5F3759DF) - (i >> 1),
                                 jnp.float32)
    for _ in range(3):
        y = y * (1.5 - 0.5 * v * y * y)
    return y


def _sc_body(nodes, chunks, xlo, xhi, row_h, col_h, ew_h, b_h, olo, ohi,
             dis_v, rows_v, row_v, col_v, ew_v, norm_v, zb_v, bvec_v,
             out_s, deg_s, sem):
    cid = lax.axis_index("c")
    sid = lax.axis_index("s")
    ebase = sid * (chunks * _CH)
    stripe = 640
    last = nodes - 15 * stripe
    nbase = pl.multiple_of(sid * stripe, 8)

    @pl.when(sid == 0)
    def _():
        for g in range(128):
            zb_v[pl.ds(g * 16, 16)] = jnp.zeros((16,), jnp.float32)
        full, rem = divmod(nodes, 2048)
        for i in range(full):
            pltpu.sync_copy(zb_v, deg_s.at[pl.ds(i * 2048, 2048)])
        if rem:
            pltpu.sync_copy(zb_v.at[pl.ds(0, rem)],
                            deg_s.at[pl.ds(full * 2048, rem)])

    plsc.subcore_barrier()

    def deg_body(ci, c):
        base = ebase + ci * _CH
        pltpu.sync_copy(col_h.at[pl.ds(base, _CH)], col_v)
        pltpu.sync_copy(ew_h.at[pl.ds(base, _CH)], ew_v)
        pltpu.sync_copy(ew_v, deg_s.at[col_v], add=True)
        return c

    lax.fori_loop(0, chunks, deg_body, 0)

    pltpu.sync_copy(b_h.at[pl.ds(cid * 128, 128)], bvec_v)
    bvals = [bvec_v[pl.ds(g * 16, 16)] for g in range(8)]

    def bfill(j, c):
        for g in range(8):
            rows_v[j, pl.ds(g * 16, 16)] = bvals[g]
        return c

    lax.fori_loop(0, _CH, bfill, 0)
    def init_stripe(count):
        full, rem = divmod(count, _CH)
        for i in range(full):
            pltpu.sync_copy(rows_v, out_s.at[pl.ds(nbase + i * _CH, _CH), :])
        if rem:
            pltpu.sync_copy(rows_v.at[pl.ds(0, rem), :],
                            out_s.at[pl.ds(nbase + full * _CH, rem), :])

    @pl.when(sid < 15)
    def _():
        init_stripe(stripe)

    @pl.when(sid == 15)
    def _():
        init_stripe(last)

    plsc.subcore_barrier()

    pltpu.sync_copy(deg_s, dis_v)

    def dis_body(g, c):
        v = dis_v[pl.ds(g * 16, 16)]
        dis_v[pl.ds(g * 16, 16)] = _fast_rsqrt(v)
        return c

    lax.fori_loop(0, nodes // 16, dis_body, 0)

    def run_edges(xtab):
        def mbody(ci, c):
            base = ebase + ci * _CH
            pltpu.sync_copy(row_h.at[pl.ds(base, _CH)], row_v)
            pltpu.sync_copy(col_h.at[pl.ds(base, _CH)], col_v)
            pltpu.sync_copy(ew_h.at[pl.ds(base, _CH)], ew_v)
            gat = pltpu.async_copy(xtab.at[row_v], rows_v, sem)
            for g in range(_CH // 16):
                r = row_v[pl.ds(g * 16, 16)]
                cc = col_v[pl.ds(g * 16, 16)]
                w = ew_v[pl.ds(g * 16, 16)]
                dr = plsc.load_gather(dis_v, [r])
                dc = plsc.load_gather(dis_v, [cc])
                norm_v[pl.ds(g * 16, 16)] = dr * w * dc
            gat.wait()

            def sbody(j, c2):
                nv = plsc.load_gather(
                    norm_v, [jnp.full((16,), j, jnp.int32)])
                for g in range(8):
                    rows_v[j, pl.ds(g * 16, 16)] = (
                        rows_v[j, pl.ds(g * 16, 16)] * nv)
                return c2

            lax.fori_loop(0, _CH, sbody, 0)
            pltpu.sync_copy(rows_v, out_s.at[col_v], add=True)
            return c

        lax.fori_loop(0, chunks, mbody, 0)

    @pl.when(cid == 0)
    def _():
        run_edges(xlo)

    @pl.when(cid == 1)
    def _():
        run_edges(xhi)

    plsc.subcore_barrier()

    def wout(otab, count):
        pltpu.sync_copy(out_s.at[pl.ds(nbase, count)],
                        otab.at[pl.ds(nbase, count)])

    for which, otab in ((0, olo), (1, ohi)):
        @pl.when((cid == which) & (sid < 15))
        def _(otab=otab):
            wout(otab, stripe)

        @pl.when((cid == which) & (sid == 15))
        def _(otab=otab):
            wout(otab, last)


def _aggregate(xlo, xhi, row2, col2, ew2, b_lin, nodes, chunks):
    dh = xlo.shape[1]
    mesh = plsc.VectorSubcoreMesh(core_axis_name="c", subcore_axis_name="s")
    out = jax.ShapeDtypeStruct((nodes, dh), jnp.float32)
    k = pl.kernel(
        functools.partial(_sc_body, nodes, chunks),
        out_type=(out, out),
        mesh=mesh,
        compiler_params=pltpu.CompilerParams(needs_layout_passes=False),
        scratch_types=[
            pltpu.VMEM((nodes,), jnp.float32),
            pltpu.VMEM((_CH, dh), jnp.float32),
            pltpu.VMEM((_CH,), jnp.int32),
            pltpu.VMEM((_CH,), jnp.int32),
            pltpu.VMEM((_CH,), jnp.float32),
            pltpu.VMEM((_CH,), jnp.float32),
            pltpu.VMEM((2048,), jnp.float32),
            pltpu.VMEM((dh,), jnp.float32),
            pltpu.VMEM_SHARED((nodes, dh), jnp.float32),
            pltpu.VMEM_SHARED((nodes,), jnp.float32),
            pltpu.SemaphoreType.DMA,
        ],
    )
    return k(xlo, xhi, row2, col2, ew2, b_lin)


def kernel(x, edge_index, edge_weight, W_lin, b_lin, W_u, U_u, b_u,
           W_r, U_r, b_r, W_h, U_h, b_h):
    n, d = x.shape
    e = edge_index.shape[1]

    xlo, xhi = _project(x, W_lin, W_u, U_u, b_u, W_r, U_r, b_r, W_h, U_h, b_h)

    e2 = e + n
    chunks = -(-e2 // (16 * _CH))
    pad = chunks * 16 * _CH - e2
    loop = jnp.arange(n, dtype=edge_index.dtype)
    zpad_i = jnp.zeros((pad,), edge_index.dtype)
    row2 = jnp.concatenate([edge_index[0], loop, zpad_i])
    col2 = jnp.concatenate([edge_index[1], loop, zpad_i])
    ew2 = jnp.concatenate([edge_weight, jnp.ones((n,), edge_weight.dtype),
                           jnp.zeros((pad,), edge_weight.dtype)])

    olo, ohi = _aggregate(xlo, xhi, row2, col2, ew2, b_lin, n, chunks)
    return jnp.concatenate([olo, ohi], axis=1)


# speedup vs baseline: 10.2943x; 1.5770x over previous
import functools

import jax
import jax.numpy as jnp
from jax import lax
from jax.experimental import pallas as pl
from jax.experimental.pallas import tpu as pltpu
from jax.experimental.pallas import tpu_sc as plsc


def _tc_body(x_ref, wl_ref, wu_ref, uu_ref, bu_ref, wr_ref, ur_ref, br_ref,
             wh_ref, uh_ref, bh_ref, lo_ref, hi_ref, wp_ref):
    @pl.when(pl.program_id(0) == 0)
    def _():
        w = wl_ref[...]

        def mm(a, b):
            return jnp.dot(a, b, preferred_element_type=jnp.float32)

        def sig(z):
            return 1.0 / (1.0 + jnp.exp(-z))

        upd = sig(mm(wu_ref[...], w) + mm(uu_ref[...], w) + bu_ref[...])
        rst = sig(mm(wr_ref[...], w) + mm(ur_ref[...], w) + br_ref[...])
        hc = jnp.tanh(mm(wh_ref[...], w) + mm(uh_ref[...], rst * w)
                      + bh_ref[...])
        wp_ref[...] = (1.0 - upd) * w + upd * hc

    y = lax.dot_general(x_ref[...], wp_ref[...],
                        dimension_numbers=(((1,), (1,)), ((), ())),
                        preferred_element_type=jnp.float32)
    lo_ref[...] = y[:, :128]
    hi_ref[...] = y[:, 128:]


def _project(x, w_lin, w_u, u_u, b_u, w_r, u_r, b_r, w_h, u_h, b_h):
    n, d = x.shape
    blk = 1000
    grid = n // blk
    wspec = pl.BlockSpec((d, d), lambda i: (0, 0))
    return pl.pallas_call(
        _tc_body,
        grid=(grid,),
        in_specs=[pl.BlockSpec((blk, d), lambda i: (i, 0))] + [wspec] * 10,
        out_specs=[pl.BlockSpec((blk, d // 2), lambda i: (i, 0))] * 2,
        out_shape=[jax.ShapeDtypeStruct((n, d // 2), jnp.float32)] * 2,
        scratch_shapes=[pltpu.VMEM((d, d), jnp.float32)],
    )(x, w_lin, w_u, u_u, b_u, w_r, u_r, b_r, w_h, u_h, b_h)


_CH = 128


def _fast_rsqrt(v):
    i = lax.bitcast_convert_type(v, jnp.int32)
    y = lax.bitcast_convert_type(jnp.int32(0x5F3759DF) - (i >> 1),
                                 jnp.float32)
    for _ in range(3):
        y = y * (1.5 - 0.5 * v * y * y)
    return y


def _sc_body(nodes, chunks, xlo, xhi, pk_h, b_h, olo, ohi,
             dis_v, rows0, rows1, eb0, eb1, col_v, ew_v, norm_v,
             zb_v, bvec_v, out_s, deg_s, es0, es1, gs0, gs1):
    cid = lax.axis_index("c")
    sid = lax.axis_index("s")
    ebufs = (eb0, eb1)
    rows = (rows0, rows1)
    esems = (es0, es1)
    gsems = (gs0, gs1)
    cbase = sid * chunks
    stripe = 640
    last = nodes - 15 * stripe
    nbase = pl.multiple_of(sid * stripe, 8)

    def rowcopy(eb, srow, dst_v, cast):
        for g in range(8):
            v = eb[srow, pl.ds(g * 16, 16)]
            if cast:
                v = lax.bitcast_convert_type(v, jnp.float32)
            dst_v[pl.ds(g * 16, 16)] = v

    @pl.when(sid == 0)
    def _():
        for g in range(128):
            zb_v[pl.ds(g * 16, 16)] = jnp.zeros((16,), jnp.float32)
        full, rem = divmod(nodes, 2048)
        for i in range(full):
            pltpu.sync_copy(zb_v, deg_s.at[pl.ds(i * 2048, 2048)])
        if rem:
            pltpu.sync_copy(zb_v.at[pl.ds(0, rem)],
                            deg_s.at[pl.ds(full * 2048, rem)])

    plsc.subcore_barrier()

    def deg_step(cur, gci, nxt_gci):
        eb, nb = ebufs[cur], ebufs[1 - cur]
        if nxt_gci is not None:
            pltpu.async_copy(pk_h.at[nxt_gci], nb, esems[1 - cur])
        rowcopy(eb, 1, col_v, cast=False)
        rowcopy(eb, 2, ew_v, cast=True)
        pltpu.sync_copy(ew_v, deg_s.at[col_v], add=True)
        if nxt_gci is not None:
            pltpu.make_async_copy(pk_h.at[nxt_gci], nb,
                                  esems[1 - cur]).wait()

    pltpu.sync_copy(pk_h.at[cbase], ebufs[0])

    def deg_pair(p, c):
        gci = cbase + 2 * p
        deg_step(0, gci, gci + 1)
        deg_step(1, gci + 1, gci + 2)
        return c

    lax.fori_loop(0, chunks // 2 - 1, deg_pair, 0)
    gci = cbase + chunks - 2
    deg_step(0, gci, gci + 1)
    deg_step(1, gci + 1, None)

    pltpu.sync_copy(b_h.at[pl.ds(cid * 128, 128)], bvec_v)
    bvals = [bvec_v[pl.ds(g * 16, 16)] for g in range(8)]

    def bfill(j, c):
        for g in range(8):
            rows0[j, pl.ds(g * 16, 16)] = bvals[g]
        return c

    lax.fori_loop(0, _CH, bfill, 0)

    def init_stripe(count):
        full, rem = divmod(count, _CH)
        for i in range(full):
            pltpu.sync_copy(rows0, out_s.at[pl.ds(nbase + i * _CH, _CH), :])
        if rem:
            pltpu.sync_copy(rows0.at[pl.ds(0, rem), :],
                            out_s.at[pl.ds(nbase + full * _CH, rem), :])

    @pl.when(sid < 15)
    def _():
        init_stripe(stripe)

    @pl.when(sid == 15)
    def _():
        init_stripe(last)

    plsc.subcore_barrier()

    pltpu.sync_copy(deg_s, dis_v)

    def dis_body(g, c):
        v = dis_v[pl.ds(g * 16, 16)]
        dis_v[pl.ds(g * 16, 16)] = _fast_rsqrt(v)
        return c

    lax.fori_loop(0, nodes // 16, dis_body, 0)

    def run_edges(xtab):
        def step(cur, gci, nxt_gci):
            eb, nb = ebufs[cur], ebufs[1 - cur]
            rc, rn = rows[cur], rows[1 - cur]
            if nxt_gci is not None:
                pltpu.async_copy(pk_h.at[nxt_gci], nb, esems[1 - cur])
            for g in range(8):
                sl = pl.ds(g * 16, 16)
                r = eb[0, sl]
                cc = eb[1, sl]
                w = lax.bitcast_convert_type(eb[2, sl], jnp.float32)
                dr = plsc.load_gather(dis_v, [r])
                dc = plsc.load_gather(dis_v, [cc])
                norm_v[sl] = dr * w * dc
            rowcopy(eb, 1, col_v, cast=False)
            pltpu.make_async_copy(xtab.at[eb.at[0]], rc, gsems[cur]).wait()
            if nxt_gci is not None:
                pltpu.make_async_copy(pk_h.at[nxt_gci], nb,
                                      esems[1 - cur]).wait()
                pltpu.async_copy(xtab.at[nb.at[0]], rn, gsems[1 - cur])

            def s4(jj, c2):
                for k in range(4):
                    j = jj * 4 + k
                    nv = plsc.load_gather(
                        norm_v, [jnp.full((16,), j, jnp.int32)])
                    for g in range(8):
                        sl = pl.ds(g * 16, 16)
                        rc[j, sl] = rc[j, sl] * nv
                return c2

            lax.fori_loop(0, _CH // 4, s4, 0)
            pltpu.sync_copy(rc, out_s.at[col_v], add=True)

        pltpu.sync_copy(pk_h.at[cbase], ebufs[0])
        pltpu.async_copy(xtab.at[ebufs[0].at[0]], rows[0], gsems[0])

        def pair(p, c):
            gci = cbase + 2 * p
            step(0, gci, gci + 1)
            step(1, gci + 1, gci + 2)
            return c

        lax.fori_loop(0, chunks // 2 - 1, pair, 0)
        gci = cbase + chunks - 2
        step(0, gci, gci + 1)
        step(1, gci + 1, None)

    @pl.when(cid == 0)
    def _():
        run_edges(xlo)

    @pl.when(cid == 1)
    def _():
        run_edges(xhi)

    plsc.subcore_barrier()

    def wout(otab, count):
        pltpu.sync_copy(out_s.at[pl.ds(nbase, count)],
                        otab.at[pl.ds(nbase, count)])

    for which, otab in ((0, olo), (1, ohi)):
        @pl.when((cid == which) & (sid < 15))
        def _(otab=otab):
            wout(otab, stripe)

        @pl.when((cid == which) & (sid == 15))
        def _(otab=otab):
            wout(otab, last)


def _aggregate(xlo, xhi, packed, b_lin, nodes, chunks):
    dh = xlo.shape[1]
    mesh = plsc.VectorSubcoreMesh(core_axis_name="c", subcore_axis_name="s")
    out = jax.ShapeDtypeStruct((nodes, dh), jnp.float32)
    k = pl.kernel(
        functools.partial(_sc_body, nodes, chunks),
        out_type=(out, out),
        mesh=mesh,
        compiler_params=pltpu.CompilerParams(needs_layout_passes=False),
        scratch_types=[
            pltpu.VMEM((nodes,), jnp.float32),
            pltpu.VMEM((_CH, dh), jnp.float32),
            pltpu.VMEM((_CH, dh), jnp.float32),
            pltpu.VMEM((3, _CH), jnp.int32),
            pltpu.VMEM((3, _CH), jnp.int32),
            pltpu.VMEM((_CH,), jnp.int32),
            pltpu.VMEM((_CH,), jnp.float32),
            pltpu.VMEM((_CH,), jnp.float32),
            pltpu.VMEM((2048,), jnp.float32),
            pltpu.VMEM((dh,), jnp.float32),
            pltpu.VMEM_SHARED((nodes, dh), jnp.float32),
            pltpu.VMEM_SHARED((nodes,), jnp.float32),
            pltpu.SemaphoreType.DMA,
            pltpu.SemaphoreType.DMA,
            pltpu.SemaphoreType.DMA,
            pltpu.SemaphoreType.DMA,
        ],
    )
    return k(xlo, xhi, packed, b_lin)


def kernel(x, edge_index, edge_weight, W_lin, b_lin, W_u, U_u, b_u,
           W_r, U_r, b_r, W_h, U_h, b_h):
    n, d = x.shape
    e = edge_index.shape[1]

    xlo, xhi = _project(x, W_lin, W_u, U_u, b_u, W_r, U_r, b_r, W_h, U_h, b_h)

    e2 = e + n
    chunks = -(-e2 // (16 * _CH))
    if chunks % 2:
        chunks += 1
    tot = chunks * 16
    pad = tot * _CH - e2
    loop = jnp.arange(n, dtype=edge_index.dtype)
    zpad_i = jnp.zeros((pad,), edge_index.dtype)
    row2 = jnp.concatenate([edge_index[0], loop, zpad_i])
    col2 = jnp.concatenate([edge_index[1], loop, zpad_i])
    ew2 = jnp.concatenate([edge_weight, jnp.ones((n,), edge_weight.dtype),
                           jnp.zeros((pad,), edge_weight.dtype)])
    ew_bits = lax.bitcast_convert_type(ew2, jnp.int32)
    packed = jnp.stack([row2.reshape(tot, _CH), col2.reshape(tot, _CH),
                        ew_bits.reshape(tot, _CH)], axis=1)

    olo, ohi = _aggregate(xlo, xhi, packed, b_lin, n, chunks)
    return jnp.concatenate([olo, ohi], axis=1)


# speedup vs baseline: 10.5273x; 1.0226x over previous
import functools

import jax
import jax.numpy as jnp
from jax import lax
from jax.experimental import pallas as pl
from jax.experimental.pallas import tpu as pltpu
from jax.experimental.pallas import tpu_sc as plsc


def _tc_body(x_ref, wl_ref, wu_ref, uu_ref, bu_ref, wr_ref, ur_ref, br_ref,
             wh_ref, uh_ref, bh_ref, lo_ref, hi_ref, wp_ref):
    @pl.when(pl.program_id(0) == 0)
    def _():
        w = wl_ref[...]

        def mm(a, b):
            return jnp.dot(a, b, preferred_element_type=jnp.float32)

        def sig(z):
            return 1.0 / (1.0 + jnp.exp(-z))

        upd = sig(mm(wu_ref[...], w) + mm(uu_ref[...], w) + bu_ref[...])
        rst = sig(mm(wr_ref[...], w) + mm(ur_ref[...], w) + br_ref[...])
        hc = jnp.tanh(mm(wh_ref[...], w) + mm(uh_ref[...], rst * w)
                      + bh_ref[...])
        wp_ref[...] = (1.0 - upd) * w + upd * hc

    y = lax.dot_general(x_ref[...], wp_ref[...],
                        dimension_numbers=(((1,), (1,)), ((), ())),
                        preferred_element_type=jnp.float32)
    lo_ref[...] = y[:, :128]
    hi_ref[...] = y[:, 128:]


def _project(x, w_lin, w_u, u_u, b_u, w_r, u_r, b_r, w_h, u_h, b_h):
    n, d = x.shape
    blk = 1000
    grid = n // blk
    wspec = pl.BlockSpec((d, d), lambda i: (0, 0))
    return pl.pallas_call(
        _tc_body,
        grid=(grid,),
        in_specs=[pl.BlockSpec((blk, d), lambda i: (i, 0))] + [wspec] * 10,
        out_specs=[pl.BlockSpec((blk, d // 2), lambda i: (i, 0))] * 2,
        out_shape=[jax.ShapeDtypeStruct((n, d // 2), jnp.float32)] * 2,
        scratch_shapes=[pltpu.VMEM((d, d), jnp.float32)],
    )(x, w_lin, w_u, u_u, b_u, w_r, u_r, b_r, w_h, u_h, b_h)


_CH = 128


def _fast_rsqrt(v):
    i = lax.bitcast_convert_type(v, jnp.int32)
    y = lax.bitcast_convert_type(jnp.int32(0x5F3759DF) - (i >> 1),
                                 jnp.float32)
    for _ in range(3):
        y = y * (1.5 - 0.5 * v * y * y)
    return y


def _sc_body(nodes, chunks, xlo, xhi, pk_h, b_h, olo, ohi,
             dis_v, rows0, rows1, eb0, eb1, col_v, ew_v, norm_v,
             zb_v, bvec_v, out_s, deg_s, es0, es1, gs0, gs1):
    cid = lax.axis_index("c")
    sid = lax.axis_index("s")
    ebufs = (eb0, eb1)
    rows = (rows0, rows1)
    esems = (es0, es1)
    gsems = (gs0, gs1)
    cbase = sid * chunks
    stripe = 640
    last = nodes - 15 * stripe
    nbase = pl.multiple_of(sid * stripe, 8)

    def rowcopy(eb, srow, dst_v, cast):
        for g in range(8):
            v = eb[srow, pl.ds(g * 16, 16)]
            if cast:
                v = lax.bitcast_convert_type(v, jnp.float32)
            dst_v[pl.ds(g * 16, 16)] = v

    @pl.when(sid == 0)
    def _():
        for g in range(128):
            zb_v[pl.ds(g * 16, 16)] = jnp.zeros((16,), jnp.float32)
        full, rem = divmod(nodes, 2048)
        for i in range(full):
            pltpu.sync_copy(zb_v, deg_s.at[pl.ds(i * 2048, 2048)])
        if rem:
            pltpu.sync_copy(zb_v.at[pl.ds(0, rem)],
                            deg_s.at[pl.ds(full * 2048, rem)])

    plsc.subcore_barrier()

    def deg_step(cur, gci, nxt_gci):
        eb, nb = ebufs[cur], ebufs[1 - cur]
        if nxt_gci is not None:
            pltpu.async_copy(pk_h.at[nxt_gci], nb, esems[1 - cur])
        rowcopy(eb, 1, col_v, cast=False)
        rowcopy(eb, 2, ew_v, cast=True)
        pltpu.sync_copy(ew_v, deg_s.at[col_v], add=True)
        if nxt_gci is not None:
            pltpu.make_async_copy(pk_h.at[nxt_gci], nb,
                                  esems[1 - cur]).wait()

    with jax.named_scope("deg_phase"):
        pltpu.sync_copy(pk_h.at[cbase], ebufs[0])

        def deg_pair(p, c):
            gci = cbase + 2 * p
            deg_step(0, gci, gci + 1)
            deg_step(1, gci + 1, gci + 2)
            return c

        lax.fori_loop(0, chunks // 2 - 1, deg_pair, 0)
        gci = cbase + chunks - 2
        deg_step(0, gci, gci + 1)
        deg_step(1, gci + 1, None)

    pltpu.sync_copy(b_h.at[pl.ds(cid * 128, 128)], bvec_v)
    bvals = [bvec_v[pl.ds(g * 16, 16)] for g in range(8)]

    def bfill(j, c):
        for g in range(8):
            rows0[j, pl.ds(g * 16, 16)] = bvals[g]
        return c

    lax.fori_loop(0, _CH, bfill, 0)

    def init_stripe(count):
        full, rem = divmod(count, _CH)
        for i in range(full):
            pltpu.sync_copy(rows0, out_s.at[pl.ds(nbase + i * _CH, _CH), :])
        if rem:
            pltpu.sync_copy(rows0.at[pl.ds(0, rem), :],
                            out_s.at[pl.ds(nbase + full * _CH, rem), :])

    @pl.when(sid < 15)
    def _():
        init_stripe(stripe)

    @pl.when(sid == 15)
    def _():
        init_stripe(last)

    plsc.subcore_barrier()

    with jax.named_scope("dis_phase"):
        pltpu.sync_copy(deg_s, dis_v)

        def dis_body(g, c):
            v = dis_v[pl.ds(g * 16, 16)]
            dis_v[pl.ds(g * 16, 16)] = _fast_rsqrt(v)
            return c

        lax.fori_loop(0, nodes // 16, dis_body, 0)

    def run_edges(xtab):
        def step(cur, gci, nxt_gci):
            eb, nb = ebufs[cur], ebufs[1 - cur]
            rc, rn = rows[cur], rows[1 - cur]
            if nxt_gci is not None:
                pltpu.async_copy(pk_h.at[nxt_gci], nb, esems[1 - cur])
            for g in range(8):
                sl = pl.ds(g * 16, 16)
                r = eb[0, sl]
                cc = eb[1, sl]
                w = lax.bitcast_convert_type(eb[2, sl], jnp.float32)
                dr = plsc.load_gather(dis_v, [r])
                dc = plsc.load_gather(dis_v, [cc])
                norm_v[sl] = dr * w * dc
            rowcopy(eb, 1, col_v, cast=False)
            pltpu.make_async_copy(xtab.at[eb.at[0]], rc, gsems[cur]).wait()
            if nxt_gci is not None:
                pltpu.make_async_copy(pk_h.at[nxt_gci], nb,
                                      esems[1 - cur]).wait()
                pltpu.async_copy(xtab.at[nb.at[0]], rn, gsems[1 - cur])

            def s4(jj, c2):
                for k in range(4):
                    j = jj * 4 + k
                    nv = plsc.load_gather(
                        norm_v, [jnp.full((16,), j, jnp.int32)])
                    for g in range(8):
                        sl = pl.ds(g * 16, 16)
                        rc[j, sl] = rc[j, sl] * nv
                return c2

            lax.fori_loop(0, _CH // 4, s4, 0)
            pltpu.sync_copy(rc, out_s.at[col_v], add=True)

        with jax.named_scope("edge_phase"):
            pltpu.sync_copy(pk_h.at[cbase], ebufs[0])
            pltpu.async_copy(xtab.at[ebufs[0].at[0]], rows[0], gsems[0])

            def pair(p, c):
                gci = cbase + 2 * p
                step(0, gci, gci + 1)
                step(1, gci + 1, gci + 2)
                return c

            lax.fori_loop(0, chunks // 2 - 1, pair, 0)
            gci = cbase + chunks - 2
            step(0, gci, gci + 1)
            step(1, gci + 1, None)

    @pl.when(cid == 0)
    def _():
        run_edges(xlo)

    @pl.when(cid == 1)
    def _():
        run_edges(xhi)

    plsc.subcore_barrier()

    def wout(otab, count):
        pltpu.sync_copy(out_s.at[pl.ds(nbase, count)],
                        otab.at[pl.ds(nbase, count)])

    for which, otab in ((0, olo), (1, ohi)):
        @pl.when((cid == which) & (sid < 15))
        def _(otab=otab):
            wout(otab, stripe)

        @pl.when((cid == which) & (sid == 15))
        def _(otab=otab):
            wout(otab, last)


def _aggregate(xlo, xhi, packed, b_lin, nodes, chunks):
    dh = xlo.shape[1]
    mesh = plsc.VectorSubcoreMesh(core_axis_name="c", subcore_axis_name="s")
    out = jax.ShapeDtypeStruct((nodes, dh), jnp.float32)
    k = pl.kernel(
        functools.partial(_sc_body, nodes, chunks),
        out_type=(out, out),
        mesh=mesh,
        compiler_params=pltpu.CompilerParams(needs_layout_passes=False),
        scratch_types=[
            pltpu.VMEM((nodes,), jnp.float32),
            pltpu.VMEM((_CH, dh), jnp.float32),
            pltpu.VMEM((_CH, dh), jnp.float32),
            pltpu.VMEM((3, _CH), jnp.int32),
            pltpu.VMEM((3, _CH), jnp.int32),
            pltpu.VMEM((_CH,), jnp.int32),
            pltpu.VMEM((_CH,), jnp.float32),
            pltpu.VMEM((_CH,), jnp.float32),
            pltpu.VMEM((2048,), jnp.float32),
            pltpu.VMEM((dh,), jnp.float32),
            pltpu.VMEM_SHARED((nodes, dh), jnp.float32),
            pltpu.VMEM_SHARED((nodes,), jnp.float32),
            pltpu.SemaphoreType.DMA,
            pltpu.SemaphoreType.DMA,
            pltpu.SemaphoreType.DMA,
            pltpu.SemaphoreType.DMA,
        ],
    )
    return k(xlo, xhi, packed, b_lin)


def kernel(x, edge_index, edge_weight, W_lin, b_lin, W_u, U_u, b_u,
           W_r, U_r, b_r, W_h, U_h, b_h):
    n, d = x.shape
    e = edge_index.shape[1]

    xlo, xhi = _project(x, W_lin, W_u, U_u, b_u, W_r, U_r, b_r, W_h, U_h, b_h)

    e2 = e + n
    chunks = -(-e2 // (16 * _CH))
    if chunks % 2:
        chunks += 1
    tot = chunks * 16
    pad = tot * _CH - e2
    loop = jnp.arange(n, dtype=edge_index.dtype)
    zpad_i = jnp.zeros((pad,), edge_index.dtype)
    row2 = jnp.concatenate([edge_index[0], loop, zpad_i])
    col2 = jnp.concatenate([edge_index[1], loop, zpad_i])
    ew2 = jnp.concatenate([edge_weight, jnp.ones((n,), edge_weight.dtype),
                           jnp.zeros((pad,), edge_weight.dtype)])
    ew_bits = lax.bitcast_convert_type(ew2, jnp.int32)
    packed = jnp.stack([row2.reshape(tot, _CH), col2.reshape(tot, _CH),
                        ew_bits.reshape(tot, _CH)], axis=1)

    olo, ohi = _aggregate(xlo, xhi, packed, b_lin, n, chunks)
    return jnp.concatenate([olo, ohi], axis=1)


# speedup vs baseline: 11.4776x; 1.0903x over previous
import functools

import jax
import jax.numpy as jnp
from jax import lax
from jax.experimental import pallas as pl
from jax.experimental.pallas import tpu as pltpu
from jax.experimental.pallas import tpu_sc as plsc


def _tc_body(x_ref, wl_ref, wu_ref, uu_ref, bu_ref, wr_ref, ur_ref, br_ref,
             wh_ref, uh_ref, bh_ref, lo_ref, hi_ref, wp_ref):
    @pl.when(pl.program_id(0) == 0)
    def _():
        w = wl_ref[...]

        def mm(a, b):
            return jnp.dot(a, b, preferred_element_type=jnp.float32)

        def sig(z):
            return 1.0 / (1.0 + jnp.exp(-z))

        upd = sig(mm(wu_ref[...], w) + mm(uu_ref[...], w) + bu_ref[...])
        rst = sig(mm(wr_ref[...], w) + mm(ur_ref[...], w) + br_ref[...])
        hc = jnp.tanh(mm(wh_ref[...], w) + mm(uh_ref[...], rst * w)
                      + bh_ref[...])
        wp_ref[...] = (1.0 - upd) * w + upd * hc

    y = lax.dot_general(x_ref[...], wp_ref[...],
                        dimension_numbers=(((1,), (1,)), ((), ())),
                        preferred_element_type=jnp.float32)
    lo_ref[...] = y[:, :128]
    hi_ref[...] = y[:, 128:]


def _project(x, w_lin, w_u, u_u, b_u, w_r, u_r, b_r, w_h, u_h, b_h):
    n, d = x.shape
    blk = 1000
    grid = n // blk
    wspec = pl.BlockSpec((d, d), lambda i: (0, 0))
    return pl.pallas_call(
        _tc_body,
        grid=(grid,),
        in_specs=[pl.BlockSpec((blk, d), lambda i: (i, 0))] + [wspec] * 10,
        out_specs=[pl.BlockSpec((blk, d // 2), lambda i: (i, 0))] * 2,
        out_shape=[jax.ShapeDtypeStruct((n, d // 2), jnp.float32)] * 2,
        scratch_shapes=[pltpu.VMEM((d, d), jnp.float32)],
    )(x, w_lin, w_u, u_u, b_u, w_r, u_r, b_r, w_h, u_h, b_h)


_CH = 128


def _fast_rsqrt(v):
    i = lax.bitcast_convert_type(v, jnp.int32)
    y = lax.bitcast_convert_type(jnp.int32(0x5F3759DF) - (i >> 1),
                                 jnp.float32)
    for _ in range(3):
        y = y * (1.5 - 0.5 * v * y * y)
    return y


def _sc_body(nodes, chunks, xlo, xhi, pk_h, b_h, olo, ohi,
             dis_v, rows0, rows1, eb0, eb1, col0, col1, ew0, ew1, norm_v,
             zb_v, bvec_v, out_s, deg_s, es0, es1, gs0, gs1, ss0, ss1):
    cid = lax.axis_index("c")
    sid = lax.axis_index("s")
    ebufs = (eb0, eb1)
    rows = (rows0, rows1)
    cols = (col0, col1)
    ews = (ew0, ew1)
    esems = (es0, es1)
    gsems = (gs0, gs1)
    ssems = (ss0, ss1)
    cbase = sid * chunks
    stripe = 640
    last = nodes - 15 * stripe
    nbase = pl.multiple_of(sid * stripe, 8)

    def rowcopy(eb, srow, dst_v, cast):
        for g in range(8):
            v = eb[srow, pl.ds(g * 16, 16)]
            if cast:
                v = lax.bitcast_convert_type(v, jnp.float32)
            dst_v[pl.ds(g * 16, 16)] = v

    for g in range(40):
        zb_v[pl.ds(g * 16, 16)] = jnp.zeros((16,), jnp.float32)

    @pl.when(sid < 15)
    def _():
        pltpu.sync_copy(zb_v, deg_s.at[pl.ds(nbase, stripe)])

    @pl.when(sid == 15)
    def _():
        pltpu.sync_copy(zb_v.at[pl.ds(0, last)],
                        deg_s.at[pl.ds(nbase, last)])

    pltpu.sync_copy(b_h.at[pl.ds(cid * 128, 128)], bvec_v)
    bvals = [bvec_v[pl.ds(g * 16, 16)] for g in range(8)]

    def bfill(j, c):
        for g in range(8):
            rows0[j, pl.ds(g * 16, 16)] = bvals[g]
        return c

    lax.fori_loop(0, _CH, bfill, 0)

    plsc.subcore_barrier()

    init_n = stripe // _CH

    def init_copies(fire):
        for i in range(init_n):
            src = rows0
            dst = out_s.at[pl.ds(nbase + i * _CH, _CH), :]
            if fire:
                pltpu.async_copy(src, dst, gsems[0])
            else:
                pltpu.make_async_copy(src, dst, gsems[0]).wait()

    def init_copies_last(fire):
        full, rem = divmod(last, _CH)
        for i in range(full):
            src = rows0
            dst = out_s.at[pl.ds(nbase + i * _CH, _CH), :]
            if fire:
                pltpu.async_copy(src, dst, gsems[0])
            else:
                pltpu.make_async_copy(src, dst, gsems[0]).wait()
        if rem:
            src = rows0.at[pl.ds(0, rem), :]
            dst = out_s.at[pl.ds(nbase + full * _CH, rem), :]
            if fire:
                pltpu.async_copy(src, dst, gsems[0])
            else:
                pltpu.make_async_copy(src, dst, gsems[0]).wait()

    for fire in (True,):
        @pl.when(sid < 15)
        def _(fire=fire):
            init_copies(fire)

        @pl.when(sid == 15)
        def _(fire=fire):
            init_copies_last(fire)

    def deg_step(cur, gci, nxt_gci, first=False):
        eb, nb = ebufs[cur], ebufs[1 - cur]
        if nxt_gci is not None:
            pltpu.async_copy(pk_h.at[nxt_gci], nb, esems[1 - cur])
        if not first:
            pltpu.make_async_copy(ews[cur], deg_s.at[cols[cur]],
                                  ssems[cur]).wait()
        rowcopy(eb, 1, cols[cur], cast=False)
        rowcopy(eb, 2, ews[cur], cast=True)
        pltpu.async_copy(ews[cur], deg_s.at[cols[cur]], ssems[cur],
                         add=True)
        if nxt_gci is not None:
            pltpu.make_async_copy(pk_h.at[nxt_gci], nb,
                                  esems[1 - cur]).wait()

    with jax.named_scope("deg_phase"):
        pltpu.sync_copy(pk_h.at[cbase], ebufs[0])
        deg_step(0, cbase, cbase + 1, first=True)
        deg_step(1, cbase + 1, cbase + 2, first=True)

        def deg_pair(p, c):
            gci = cbase + 2 * p
            deg_step(0, gci, gci + 1)
            deg_step(1, gci + 1, gci + 2)
            return c

        lax.fori_loop(1, chunks // 2 - 1, deg_pair, 0)
        gci = cbase + chunks - 2
        deg_step(0, gci, gci + 1)
        deg_step(1, gci + 1, None)
        for p in (0, 1):
            pltpu.make_async_copy(ews[p], deg_s.at[cols[p]],
                                  ssems[p]).wait()

    for fire in (False,):
        @pl.when(sid < 15)
        def _(fire=fire):
            init_copies(fire)

        @pl.when(sid == 15)
        def _(fire=fire):
            init_copies_last(fire)

    plsc.subcore_barrier()

    with jax.named_scope("dis_phase"):
        def dis_stripe(count):
            pltpu.sync_copy(deg_s.at[pl.ds(nbase, count)],
                            dis_v.at[pl.ds(0, count)])

            def dis_body(g, c):
                v = dis_v[pl.ds(g * 16, 16)]
                dis_v[pl.ds(g * 16, 16)] = _fast_rsqrt(v)
                return c

            lax.fori_loop(0, count // 16, dis_body, 0)
            pltpu.sync_copy(dis_v.at[pl.ds(0, count)],
                            deg_s.at[pl.ds(nbase, count)])

        @pl.when(sid < 15)
        def _():
            dis_stripe(stripe)

        @pl.when(sid == 15)
        def _():
            dis_stripe(last)

        plsc.subcore_barrier()
        pltpu.sync_copy(deg_s, dis_v)

    def run_edges(xtab):
        def step(cur, gci, nxt_gci, first=False):
            eb, nb = ebufs[cur], ebufs[1 - cur]
            rc, rn = rows[cur], rows[1 - cur]
            if nxt_gci is not None:
                pltpu.async_copy(pk_h.at[nxt_gci], nb, esems[1 - cur])
            for g in range(8):
                sl = pl.ds(g * 16, 16)
                r = eb[0, sl]
                cc = eb[1, sl]
                w = lax.bitcast_convert_type(eb[2, sl], jnp.float32)
                dr = plsc.load_gather(dis_v, [r])
                dc = plsc.load_gather(dis_v, [cc])
                norm_v[sl] = dr * w * dc
            rowcopy(eb, 1, cols[cur], cast=False)
            pltpu.make_async_copy(xtab.at[eb.at[0]], rc, gsems[cur]).wait()
            if nxt_gci is not None:
                pltpu.make_async_copy(pk_h.at[nxt_gci], nb,
                                      esems[1 - cur]).wait()
                if not first:
                    pltpu.make_async_copy(rn, out_s.at[cols[1 - cur]],
                                          ssems[1 - cur]).wait()
                pltpu.async_copy(xtab.at[nb.at[0]], rn, gsems[1 - cur])

            def s4(jj, c2):
                for k in range(4):
                    j = jj * 4 + k
                    nv = plsc.load_gather(
                        norm_v, [jnp.full((16,), j, jnp.int32)])
                    for g in range(8):
                        sl = pl.ds(g * 16, 16)
                        rc[j, sl] = rc[j, sl] * nv
                return c2

            lax.fori_loop(0, _CH // 4, s4, 0)
            pltpu.async_copy(rc, out_s.at[cols[cur]], ssems[cur], add=True)

        with jax.named_scope("edge_phase"):
            pltpu.sync_copy(pk_h.at[cbase], ebufs[0])
            pltpu.async_copy(xtab.at[ebufs[0].at[0]], rows[0], gsems[0])
            step(0, cbase, cbase + 1, first=True)
            step(1, cbase + 1, cbase + 2)

            def pair(p, c):
                gci = cbase + 2 * p
                step(0, gci, gci + 1)
                step(1, gci + 1, gci + 2)
                return c

            lax.fori_loop(1, chunks // 2 - 1, pair, 0)
            gci = cbase + chunks - 2
            step(0, gci, gci + 1)
            step(1, gci + 1, None)
            for p in (0, 1):
                pltpu.make_async_copy(rows[p], out_s.at[cols[p]],
                                      ssems[p]).wait()

    @pl.when(cid == 0)
    def _():
        run_edges(xlo)

    @pl.when(cid == 1)
    def _():
        run_edges(xhi)

    plsc.subcore_barrier()

    def wout(otab, count):
        pltpu.sync_copy(out_s.at[pl.ds(nbase, count)],
                        otab.at[pl.ds(nbase, count)])

    for which, otab in ((0, olo), (1, ohi)):
        @pl.when((cid == which) & (sid < 15))
        def _(otab=otab):
            wout(otab, stripe)

        @pl.when((cid == which) & (sid == 15))
        def _(otab=otab):
            wout(otab, last)


def _aggregate(xlo, xhi, packed, b_lin, nodes, chunks):
    dh = xlo.shape[1]
    mesh = plsc.VectorSubcoreMesh(core_axis_name="c", subcore_axis_name="s")
    out = jax.ShapeDtypeStruct((nodes, dh), jnp.float32)
    k = pl.kernel(
        functools.partial(_sc_body, nodes, chunks),
        out_type=(out, out),
        mesh=mesh,
        compiler_params=pltpu.CompilerParams(needs_layout_passes=False),
        scratch_types=[
            pltpu.VMEM((nodes,), jnp.float32),
            pltpu.VMEM((_CH, dh), jnp.float32),
            pltpu.VMEM((_CH, dh), jnp.float32),
            pltpu.VMEM((3, _CH), jnp.int32),
            pltpu.VMEM((3, _CH), jnp.int32),
            pltpu.VMEM((_CH,), jnp.int32),
            pltpu.VMEM((_CH,), jnp.int32),
            pltpu.VMEM((_CH,), jnp.float32),
            pltpu.VMEM((_CH,), jnp.float32),
            pltpu.VMEM((_CH,), jnp.float32),
            pltpu.VMEM((640,), jnp.float32),
            pltpu.VMEM((dh,), jnp.float32),
            pltpu.VMEM_SHARED((nodes, dh), jnp.float32),
            pltpu.VMEM_SHARED((nodes,), jnp.float32),
            pltpu.SemaphoreType.DMA,
            pltpu.SemaphoreType.DMA,
            pltpu.SemaphoreType.DMA,
            pltpu.SemaphoreType.DMA,
            pltpu.SemaphoreType.DMA,
            pltpu.SemaphoreType.DMA,
        ],
    )
    return k(xlo, xhi, packed, b_lin)


def kernel(x, edge_index, edge_weight, W_lin, b_lin, W_u, U_u, b_u,
           W_r, U_r, b_r, W_h, U_h, b_h):
    n, d = x.shape
    e = edge_index.shape[1]

    xlo, xhi = _project(x, W_lin, W_u, U_u, b_u, W_r, U_r, b_r, W_h, U_h, b_h)

    e2 = e + n
    chunks = -(-e2 // (16 * _CH))
    if chunks % 2:
        chunks += 1
    tot = chunks * 16
    pad = tot * _CH - e2
    loop = jnp.arange(n, dtype=edge_index.dtype)
    zpad_i = jnp.zeros((pad,), edge_index.dtype)
    row2 = jnp.concatenate([edge_index[0], loop, zpad_i])
    col2 = jnp.concatenate([edge_index[1], loop, zpad_i])
    ew2 = jnp.concatenate([edge_weight, jnp.ones((n,), edge_weight.dtype),
                           jnp.zeros((pad,), edge_weight.dtype)])
    ew_bits = lax.bitcast_convert_type(ew2, jnp.int32)
    packed = jnp.stack([row2.reshape(tot, _CH), col2.reshape(tot, _CH),
                        ew_bits.reshape(tot, _CH)], axis=1)

    olo, ohi = _aggregate(xlo, xhi, packed, b_lin, n, chunks)
    return jnp.concatenate([olo, ohi], axis=1)


# speedup vs baseline: 15.5239x; 1.3525x over previous
import functools

import jax
import jax.numpy as jnp
from jax import lax
from jax.experimental import pallas as pl
from jax.experimental.pallas import tpu as pltpu
from jax.experimental.pallas import tpu_sc as plsc


def _tc_body(x_ref, wl_ref, wu_ref, uu_ref, bu_ref, wr_ref, ur_ref, br_ref,
             wh_ref, uh_ref, bh_ref, lo_ref, hi_ref, wp_ref):
    @pl.when(pl.program_id(0) == 0)
    def _():
        w = wl_ref[...]

        def mm(a, b):
            return jnp.dot(a, b, preferred_element_type=jnp.float32)

        def sig(z):
            return 1.0 / (1.0 + jnp.exp(-z))

        upd = sig(mm(wu_ref[...], w) + mm(uu_ref[...], w) + bu_ref[...])
        rst = sig(mm(wr_ref[...], w) + mm(ur_ref[...], w) + br_ref[...])
        hc = jnp.tanh(mm(wh_ref[...], w) + mm(uh_ref[...], rst * w)
                      + bh_ref[...])
        wp_ref[...] = (1.0 - upd) * w + upd * hc

    y = lax.dot_general(x_ref[...], wp_ref[...],
                        dimension_numbers=(((1,), (1,)), ((), ())),
                        preferred_element_type=jnp.float32)
    lo_ref[...] = y[:, :128]
    hi_ref[...] = y[:, 128:]


def _project(x, w_lin, w_u, u_u, b_u, w_r, u_r, b_r, w_h, u_h, b_h):
    n, d = x.shape
    blk = 1000
    grid = n // blk
    wspec = pl.BlockSpec((d, d), lambda i: (0, 0))
    return pl.pallas_call(
        _tc_body,
        grid=(grid,),
        in_specs=[pl.BlockSpec((blk, d), lambda i: (i, 0))] + [wspec] * 10,
        out_specs=[pl.BlockSpec((blk, d // 2), lambda i: (i, 0))] * 2,
        out_shape=[jax.ShapeDtypeStruct((n, d // 2), jnp.float32)] * 2,
        scratch_shapes=[pltpu.VMEM((d, d), jnp.float32)],
    )(x, w_lin, w_u, u_u, b_u, w_r, u_r, b_r, w_h, u_h, b_h)


_CH = 128


def _fast_rsqrt(v):
    i = lax.bitcast_convert_type(v, jnp.int32)
    y = lax.bitcast_convert_type(jnp.int32(0x5F3759DF) - (i >> 1),
                                 jnp.float32)
    for _ in range(3):
        y = y * (1.5 - 0.5 * v * y * y)
    return y


def _sc_body(nodes, chunks, xlo, xhi, pk_h, b_h, olo, ohi,
             dis_v, rows0, rows1, eb0, eb1, col0, col1, ew0, ew1, norm_v,
             zb_v, bvec_v, out_s, deg_s, es0, es1, gs0, gs1, ss0, ss1):
    cid = lax.axis_index("c")
    sid = lax.axis_index("s")
    ebufs = (eb0, eb1)
    rows = (rows0, rows1)
    cols = (col0, col1)
    ews = (ew0, ew1)
    esems = (es0, es1)
    gsems = (gs0, gs1)
    ssems = (ss0, ss1)
    cbase = sid
    _NX = 16
    stripe = 640
    last = nodes - 15 * stripe
    nbase = pl.multiple_of(sid * stripe, 8)

    def rowcopy(eb, srow, dst_v, cast):
        for g in range(8):
            v = eb[srow, pl.ds(g * 16, 16)]
            if cast:
                v = lax.bitcast_convert_type(v, jnp.float32)
            dst_v[pl.ds(g * 16, 16)] = v

    for g in range(40):
        zb_v[pl.ds(g * 16, 16)] = jnp.zeros((16,), jnp.float32)

    @pl.when(sid < 15)
    def _():
        pltpu.sync_copy(zb_v, deg_s.at[pl.ds(nbase, stripe)])

    @pl.when(sid == 15)
    def _():
        pltpu.sync_copy(zb_v.at[pl.ds(0, last)],
                        deg_s.at[pl.ds(nbase, last)])

    pltpu.sync_copy(b_h.at[pl.ds(cid * 128, 128)], bvec_v)
    bvals = [bvec_v[pl.ds(g * 16, 16)] for g in range(8)]

    def bfill(j, c):
        for g in range(8):
            rows0[j, pl.ds(g * 16, 16)] = bvals[g]
        return c

    lax.fori_loop(0, _CH, bfill, 0)

    plsc.subcore_barrier()

    init_n = stripe // _CH

    def init_copies(fire):
        for i in range(init_n):
            src = rows0
            dst = out_s.at[pl.ds(nbase + i * _CH, _CH), :]
            if fire:
                pltpu.async_copy(src, dst, gsems[0])
            else:
                pltpu.make_async_copy(src, dst, gsems[0]).wait()

    def init_copies_last(fire):
        full, rem = divmod(last, _CH)
        for i in range(full):
            src = rows0
            dst = out_s.at[pl.ds(nbase + i * _CH, _CH), :]
            if fire:
                pltpu.async_copy(src, dst, gsems[0])
            else:
                pltpu.make_async_copy(src, dst, gsems[0]).wait()
        if rem:
            src = rows0.at[pl.ds(0, rem), :]
            dst = out_s.at[pl.ds(nbase + full * _CH, rem), :]
            if fire:
                pltpu.async_copy(src, dst, gsems[0])
            else:
                pltpu.make_async_copy(src, dst, gsems[0]).wait()

    for fire in (True,):
        @pl.when(sid < 15)
        def _(fire=fire):
            init_copies(fire)

        @pl.when(sid == 15)
        def _(fire=fire):
            init_copies_last(fire)

    def deg_step(cur, gci, nxt_gci, first=False):
        eb, nb = ebufs[cur], ebufs[1 - cur]
        if nxt_gci is not None:
            pltpu.async_copy(pk_h.at[nxt_gci], nb, esems[1 - cur])
        if not first:
            pltpu.make_async_copy(ews[cur], deg_s.at[cols[cur]],
                                  ssems[cur]).wait()
        rowcopy(eb, 1, cols[cur], cast=False)
        rowcopy(eb, 2, ews[cur], cast=True)
        pltpu.async_copy(ews[cur], deg_s.at[cols[cur]], ssems[cur],
                         add=True)
        if nxt_gci is not None:
            pltpu.make_async_copy(pk_h.at[nxt_gci], nb,
                                  esems[1 - cur]).wait()

    with jax.named_scope("deg_phase"):
        pltpu.sync_copy(pk_h.at[cbase], ebufs[0])
        deg_step(0, cbase, cbase + _NX, first=True)
        deg_step(1, cbase + _NX, cbase + 2 * _NX, first=True)

        def deg_pair(p, c):
            gci = cbase + _NX * 2 * p
            deg_step(0, gci, gci + _NX)
            deg_step(1, gci + _NX, gci + 2 * _NX)
            return c

        lax.fori_loop(1, chunks // 2 - 1, deg_pair, 0)
        gci = cbase + _NX * (chunks - 2)
        deg_step(0, gci, gci + _NX)
        deg_step(1, gci + _NX, None)
        for p in (0, 1):
            pltpu.make_async_copy(ews[p], deg_s.at[cols[p]],
                                  ssems[p]).wait()

    for fire in (False,):
        @pl.when(sid < 15)
        def _(fire=fire):
            init_copies(fire)

        @pl.when(sid == 15)
        def _(fire=fire):
            init_copies_last(fire)

    plsc.subcore_barrier()

    with jax.named_scope("dis_phase"):
        def dis_stripe(count):
            pltpu.sync_copy(deg_s.at[pl.ds(nbase, count)],
                            dis_v.at[pl.ds(0, count)])

            def dis_body(g, c):
                v = dis_v[pl.ds(g * 16, 16)]
                dis_v[pl.ds(g * 16, 16)] = _fast_rsqrt(v)
                return c

            lax.fori_loop(0, count // 16, dis_body, 0)
            pltpu.sync_copy(dis_v.at[pl.ds(0, count)],
                            deg_s.at[pl.ds(nbase, count)])

        @pl.when(sid < 15)
        def _():
            dis_stripe(stripe)

        @pl.when(sid == 15)
        def _():
            dis_stripe(last)

        plsc.subcore_barrier()
        pltpu.sync_copy(deg_s, dis_v)

    def run_edges(xtab):
        def step(cur, gci, nxt_gci, first=False):
            eb, nb = ebufs[cur], ebufs[1 - cur]
            rc, rn = rows[cur], rows[1 - cur]
            if nxt_gci is not None:
                pltpu.async_copy(pk_h.at[nxt_gci], nb, esems[1 - cur])
            for g in range(8):
                sl = pl.ds(g * 16, 16)
                r = eb[0, sl]
                cc = eb[1, sl]
                w = lax.bitcast_convert_type(eb[2, sl], jnp.float32)
                dr = plsc.load_gather(dis_v, [r])
                dc = plsc.load_gather(dis_v, [cc])
                norm_v[sl] = dr * w * dc
            rowcopy(eb, 1, cols[cur], cast=False)
            pltpu.make_async_copy(xtab.at[eb.at[0]], rc, gsems[cur]).wait()
            if nxt_gci is not None:
                pltpu.make_async_copy(pk_h.at[nxt_gci], nb,
                                      esems[1 - cur]).wait()
                if not first:
                    pltpu.make_async_copy(rn, out_s.at[cols[1 - cur]],
                                          ssems[1 - cur]).wait()
                pltpu.async_copy(xtab.at[nb.at[0]], rn, gsems[1 - cur])

            def s4(jj, c2):
                for k in range(4):
                    j = jj * 4 + k
                    nv = plsc.load_gather(
                        norm_v, [jnp.full((16,), j, jnp.int32)])
                    for g in range(8):
                        sl = pl.ds(g * 16, 16)
                        rc[j, sl] = rc[j, sl] * nv
                return c2

            lax.fori_loop(0, _CH // 4, s4, 0)
            pltpu.async_copy(rc, out_s.at[cols[cur]], ssems[cur], add=True)

        with jax.named_scope("edge_phase"):
            pltpu.sync_copy(pk_h.at[cbase], ebufs[0])
            pltpu.async_copy(xtab.at[ebufs[0].at[0]], rows[0], gsems[0])
            step(0, cbase, cbase + _NX, first=True)
            step(1, cbase + _NX, cbase + 2 * _NX)

            def pair(p, c):
                gci = cbase + _NX * 2 * p
                step(0, gci, gci + _NX)
                step(1, gci + _NX, gci + 2 * _NX)
                return c

            lax.fori_loop(1, chunks // 2 - 1, pair, 0)
            gci = cbase + _NX * (chunks - 2)
            step(0, gci, gci + _NX)
            step(1, gci + _NX, None)
            for p in (0, 1):
                pltpu.make_async_copy(rows[p], out_s.at[cols[p]],
                                      ssems[p]).wait()

    @pl.when(cid == 0)
    def _():
        run_edges(xlo)

    @pl.when(cid == 1)
    def _():
        run_edges(xhi)

    plsc.subcore_barrier()

    def wout(otab, count):
        pltpu.sync_copy(out_s.at[pl.ds(nbase, count)],
                        otab.at[pl.ds(nbase, count)])

    for which, otab in ((0, olo), (1, ohi)):
        @pl.when((cid == which) & (sid < 15))
        def _(otab=otab):
            wout(otab, stripe)

        @pl.when((cid == which) & (sid == 15))
        def _(otab=otab):
            wout(otab, last)


def _aggregate(xlo, xhi, packed, b_lin, nodes, chunks):
    dh = xlo.shape[1]
    mesh = plsc.VectorSubcoreMesh(core_axis_name="c", subcore_axis_name="s")
    out = jax.ShapeDtypeStruct((nodes, dh), jnp.float32)
    k = pl.kernel(
        functools.partial(_sc_body, nodes, chunks),
        out_type=(out, out),
        mesh=mesh,
        compiler_params=pltpu.CompilerParams(needs_layout_passes=False),
        scratch_types=[
            pltpu.VMEM((nodes,), jnp.float32),
            pltpu.VMEM((_CH, dh), jnp.float32),
            pltpu.VMEM((_CH, dh), jnp.float32),
            pltpu.VMEM((3, _CH), jnp.int32),
            pltpu.VMEM((3, _CH), jnp.int32),
            pltpu.VMEM((_CH,), jnp.int32),
            pltpu.VMEM((_CH,), jnp.int32),
            pltpu.VMEM((_CH,), jnp.float32),
            pltpu.VMEM((_CH,), jnp.float32),
            pltpu.VMEM((_CH,), jnp.float32),
            pltpu.VMEM((640,), jnp.float32),
            pltpu.VMEM((dh,), jnp.float32),
            pltpu.VMEM_SHARED((nodes, dh), jnp.float32),
            pltpu.VMEM_SHARED((nodes,), jnp.float32),
            pltpu.SemaphoreType.DMA,
            pltpu.SemaphoreType.DMA,
            pltpu.SemaphoreType.DMA,
            pltpu.SemaphoreType.DMA,
            pltpu.SemaphoreType.DMA,
            pltpu.SemaphoreType.DMA,
        ],
    )
    return k(xlo, xhi, packed, b_lin)


def kernel(x, edge_index, edge_weight, W_lin, b_lin, W_u, U_u, b_u,
           W_r, U_r, b_r, W_h, U_h, b_h):
    n, d = x.shape
    e = edge_index.shape[1]

    xlo, xhi = _project(x, W_lin, W_u, U_u, b_u, W_r, U_r, b_r, W_h, U_h, b_h)

    e2 = e + n
    chunks = -(-e2 // (16 * _CH))
    if chunks % 2:
        chunks += 1
    tot = chunks * 16
    pad = tot * _CH - e2
    loop = jnp.arange(n, dtype=edge_index.dtype)
    zpad_i = jnp.arange(pad, dtype=edge_index.dtype) % n
    row2 = jnp.concatenate([edge_index[0], loop, zpad_i])
    col2 = jnp.concatenate([edge_index[1], loop, zpad_i])
    ew2 = jnp.concatenate([edge_weight, jnp.ones((n,), edge_weight.dtype),
                           jnp.zeros((pad,), edge_weight.dtype)])
    ew_bits = lax.bitcast_convert_type(ew2, jnp.int32)
    packed = jnp.stack([row2.reshape(tot, _CH), col2.reshape(tot, _CH),
                        ew_bits.reshape(tot, _CH)], axis=1)

    olo, ohi = _aggregate(xlo, xhi, packed, b_lin, n, chunks)
    return jnp.concatenate([olo, ohi], axis=1)


# speedup vs baseline: 18.6870x; 1.2038x over previous
import functools

import jax
import jax.numpy as jnp
from jax import lax
from jax.experimental import pallas as pl
from jax.experimental.pallas import tpu as pltpu
from jax.experimental.pallas import tpu_sc as plsc


def _tc_body(x_ref, wl_ref, wu_ref, uu_ref, bu_ref, wr_ref, ur_ref, br_ref,
             wh_ref, uh_ref, bh_ref, lo_ref, hi_ref, wp_ref):
    @pl.when(pl.program_id(0) == 0)
    def _():
        w = wl_ref[...]

        def mm(a, b):
            return jnp.dot(a, b, preferred_element_type=jnp.float32)

        def sig(z):
            return 1.0 / (1.0 + jnp.exp(-z))

        upd = sig(mm(wu_ref[...], w) + mm(uu_ref[...], w) + bu_ref[...])
        rst = sig(mm(wr_ref[...], w) + mm(ur_ref[...], w) + br_ref[...])
        hc = jnp.tanh(mm(wh_ref[...], w) + mm(uh_ref[...], rst * w)
                      + bh_ref[...])
        wp_ref[...] = (1.0 - upd) * w + upd * hc

    y = lax.dot_general(x_ref[...], wp_ref[...],
                        dimension_numbers=(((1,), (1,)), ((), ())),
                        preferred_element_type=jnp.float32)
    lo_ref[...] = y[:, :128]
    hi_ref[...] = y[:, 128:]


def _project(x, w_lin, w_u, u_u, b_u, w_r, u_r, b_r, w_h, u_h, b_h):
    n, d = x.shape
    blk = 1000
    grid = n // blk
    wspec = pl.BlockSpec((d, d), lambda i: (0, 0))
    return pl.pallas_call(
        _tc_body,
        grid=(grid,),
        in_specs=[pl.BlockSpec((blk, d), lambda i: (i, 0))] + [wspec] * 10,
        out_specs=[pl.BlockSpec((blk, d // 2), lambda i: (i, 0))] * 2,
        out_shape=[jax.ShapeDtypeStruct((n, d // 2), jnp.float32)] * 2,
        scratch_shapes=[pltpu.VMEM((d, d), jnp.float32)],
    )(x, w_lin, w_u, u_u, b_u, w_r, u_r, b_r, w_h, u_h, b_h)


_CH = 128


def _fast_rsqrt(v):
    i = lax.bitcast_convert_type(v, jnp.int32)
    y = lax.bitcast_convert_type(jnp.int32(0x5F3759DF) - (i >> 1),
                                 jnp.float32)
    for _ in range(3):
        y = y * (1.5 - 0.5 * v * y * y)
    return y


def _sc_body(nodes, chunks, nedges, xlo, xhi, ei_h, ew_h, b_h, out_h,
             dis_v, rows0, rows1, rb0, rb1, cs0, cs1, et0, et1,
             col0, col1, ew0, ew1, norm_v, zb_v, bvec_v, out_s, deg_s,
             es0, es1, gs0, gs1, ss0, ss1):
    cid = lax.axis_index("c")
    sid = lax.axis_index("s")
    rows = (rows0, rows1)
    rbuf = (rb0, rb1)
    cstg = (cs0, cs1)
    estg = (et0, et1)
    cols = (col0, col1)
    ews = (ew0, ew1)
    esems = (es0, es1)
    gsems = (gs0, gs1)
    ssems = (ss0, ss1)
    er = nedges // _CH
    cbase = sid
    _NX = 16
    stripe = 640
    last = nodes - 15 * stripe
    nbase = pl.multiple_of(sid * stripe, 8)

    def vcopy(src_v, dst_v, n16=8):
        for g in range(n16):
            dst_v[pl.ds(g * 16, 16)] = src_v[pl.ds(g * 16, 16)]

    def load_or_synth(p, gci):
        real = gci < er
        ebase = pl.multiple_of(gci * _CH, _CH)

        @pl.when(real)
        def _():
            pltpu.async_copy(ei_h.at[pl.ds(ebase, _CH)], rbuf[p], esems[p])
            pltpu.async_copy(ei_h.at[pl.ds(nedges + ebase, _CH)],
                             cstg[p], esems[p])
            pltpu.async_copy(ew_h.at[pl.ds(ebase, _CH)], estg[p], esems[p])

        @pl.when(jnp.logical_not(real))
        def _():
            vb = (gci - er) * _CH
            for g in range(8):
                sl = pl.ds(g * 16, 16)
                idx = vb + g * 16 + lax.iota(jnp.int32, 16)
                valid = idx < nodes
                idxm = jnp.where(valid, idx, idx - nodes)
                rbuf[p][sl] = idxm
                cstg[p][sl] = idxm
                estg[p][sl] = jnp.where(valid, 1.0, 0.0)

    def wait_load(p, gci):
        real = gci < er
        ebase = pl.multiple_of(gci * _CH, _CH)

        @pl.when(real)
        def _():
            pltpu.make_async_copy(ei_h.at[pl.ds(ebase, _CH)], rbuf[p],
                                  esems[p]).wait()
            pltpu.make_async_copy(ei_h.at[pl.ds(nedges + ebase, _CH)],
                                  cstg[p], esems[p]).wait()
            pltpu.make_async_copy(ew_h.at[pl.ds(ebase, _CH)], estg[p],
                                  esems[p]).wait()

    for g in range(40):
        zb_v[pl.ds(g * 16, 16)] = jnp.zeros((16,), jnp.float32)

    @pl.when(sid < 15)
    def _():
        pltpu.sync_copy(zb_v, deg_s.at[pl.ds(nbase, stripe)])

    @pl.when(sid == 15)
    def _():
        pltpu.sync_copy(zb_v.at[pl.ds(0, last)],
                        deg_s.at[pl.ds(nbase, last)])

    pltpu.sync_copy(b_h.at[pl.ds(pl.multiple_of(cid * 128, 128), 128)],
                    bvec_v)
    bvals = [bvec_v[pl.ds(g * 16, 16)] for g in range(8)]

    def bfill(j, c):
        for g in range(8):
            rows0[j, pl.ds(g * 16, 16)] = bvals[g]
        return c

    lax.fori_loop(0, _CH, bfill, 0)

    plsc.subcore_barrier()

    def init_copies(count, fire):
        full, rem = divmod(count, _CH)
        blocks = [(i * _CH, _CH) for i in range(full)]
        if rem:
            blocks.append((full * _CH, rem))
        for off, cnt in blocks:
            src = rows0 if cnt == _CH else rows0.at[pl.ds(0, cnt), :]
            dst = out_s.at[pl.ds(nbase + off, cnt), :]
            if fire:
                pltpu.async_copy(src, dst, gsems[0])
            else:
                pltpu.make_async_copy(src, dst, gsems[0]).wait()

    for fire in (True,):
        @pl.when(sid < 15)
        def _(fire=fire):
            init_copies(stripe, fire)

        @pl.when(sid == 15)
        def _(fire=fire):
            init_copies(last, fire)

    def deg_step(p, gci, nxt_gci, first=False):
        if nxt_gci is not None:
            load_or_synth(1 - p, nxt_gci)
        wait_load(p, gci)
        if not first:
            pltpu.make_async_copy(ews[p], deg_s.at[cols[p]],
                                  ssems[p]).wait()
        vcopy(cstg[p], cols[p])
        vcopy(estg[p], ews[p])
        pltpu.async_copy(ews[p], deg_s.at[cols[p]], ssems[p], add=True)

    with jax.named_scope("deg_phase"):
        load_or_synth(0, cbase)
        deg_step(0, cbase, cbase + _NX, first=True)
        deg_step(1, cbase + _NX, cbase + 2 * _NX, first=True)

        def deg_pair(q, c):
            gci = cbase + _NX * 2 * q
            deg_step(0, gci, gci + _NX)
            deg_step(1, gci + _NX, gci + 2 * _NX)
            return c

        lax.fori_loop(1, chunks // 2 - 1, deg_pair, 0)
        gci = cbase + _NX * (chunks - 2)
        deg_step(0, gci, gci + _NX)
        deg_step(1, gci + _NX, None)
        for p in (0, 1):
            pltpu.make_async_copy(ews[p], deg_s.at[cols[p]],
                                  ssems[p]).wait()

    for fire in (False,):
        @pl.when(sid < 15)
        def _(fire=fire):
            init_copies(stripe, fire)

        @pl.when(sid == 15)
        def _(fire=fire):
            init_copies(last, fire)

    plsc.subcore_barrier()

    with jax.named_scope("dis_phase"):
        def dis_stripe(count):
            pltpu.sync_copy(deg_s.at[pl.ds(nbase, count)],
                            dis_v.at[pl.ds(0, count)])

            def dis_body(g, c):
                v = dis_v[pl.ds(g * 16, 16)]
                dis_v[pl.ds(g * 16, 16)] = _fast_rsqrt(v)
                return c

            lax.fori_loop(0, count // 16, dis_body, 0)
            pltpu.sync_copy(dis_v.at[pl.ds(0, count)],
                            deg_s.at[pl.ds(nbase, count)])

        @pl.when(sid < 15)
        def _():
            dis_stripe(stripe)

        @pl.when(sid == 15)
        def _():
            dis_stripe(last)

        plsc.subcore_barrier()
        pltpu.sync_copy(deg_s, dis_v)

    def run_edges(xtab):
        def step(p, gci, nxt_gci, first=False):
            rc, rn = rows[p], rows[1 - p]
            if nxt_gci is not None:
                load_or_synth(1 - p, nxt_gci)
            for g in range(8):
                sl = pl.ds(g * 16, 16)
                dr = plsc.load_gather(dis_v, [rbuf[p][sl]])
                dc = plsc.load_gather(dis_v, [cstg[p][sl]])
                norm_v[sl] = dr * estg[p][sl] * dc
            vcopy(cstg[p], cols[p])
            pltpu.make_async_copy(xtab.at[rbuf[p]], rc, gsems[p]).wait()
            if nxt_gci is not None:
                wait_load(1 - p, nxt_gci)
                if not first:
                    pltpu.make_async_copy(rn, out_s.at[cols[1 - p]],
                                          ssems[1 - p]).wait()
                pltpu.async_copy(xtab.at[rbuf[1 - p]], rn, gsems[1 - p])

            def s4(jj, c2):
                for k in range(4):
                    j = jj * 4 + k
                    nv = plsc.load_gather(
                        norm_v, [jnp.full((16,), j, jnp.int32)])
                    for g in range(8):
                        sl = pl.ds(g * 16, 16)
                        rc[j, sl] = rc[j, sl] * nv
                return c2

            lax.fori_loop(0, _CH // 4, s4, 0)
            pltpu.async_copy(rc, out_s.at[cols[p]], ssems[p], add=True)

        with jax.named_scope("edge_phase"):
            load_or_synth(0, cbase)
            wait_load(0, cbase)
            pltpu.async_copy(xtab.at[rbuf[0]], rows[0], gsems[0])
            step(0, cbase, cbase + _NX, first=True)
            step(1, cbase + _NX, cbase + 2 * _NX)

            def pair(q, c):
                gci = cbase + _NX * 2 * q
                step(0, gci, gci + _NX)
                step(1, gci + _NX, gci + 2 * _NX)
                return c

            lax.fori_loop(1, chunks // 2 - 1, pair, 0)
            gci = cbase + _NX * (chunks - 2)
            step(0, gci, gci + _NX)
            step(1, gci + _NX, None)
            for p in (0, 1):
                pltpu.make_async_copy(rows[p], out_s.at[cols[p]],
                                      ssems[p]).wait()

    @pl.when(cid == 0)
    def _():
        run_edges(xlo)

    @pl.when(cid == 1)
    def _():
        run_edges(xhi)

    plsc.subcore_barrier()

    chalf = pl.multiple_of(cid * 128, 128)

    def wout(count):
        pltpu.sync_copy(out_s.at[pl.ds(nbase, count)],
                        out_h.at[pl.ds(nbase, count), pl.ds(chalf, 128)])

    @pl.when(sid < 15)
    def _():
        wout(stripe)

    @pl.when(sid == 15)
    def _():
        wout(last)


def _aggregate(xlo, xhi, ei_flat, ew, b_lin, nodes, chunks, nedges):
    dh = xlo.shape[1]
    mesh = plsc.VectorSubcoreMesh(core_axis_name="c", subcore_axis_name="s")
    out = jax.ShapeDtypeStruct((nodes, 2 * dh), jnp.float32)
    k = pl.kernel(
        functools.partial(_sc_body, nodes, chunks, nedges),
        out_type=out,
        mesh=mesh,
        compiler_params=pltpu.CompilerParams(needs_layout_passes=False),
        scratch_types=[
            pltpu.VMEM((nodes,), jnp.float32),
            pltpu.VMEM((_CH, dh), jnp.float32),
            pltpu.VMEM((_CH, dh), jnp.float32),
            pltpu.VMEM((_CH,), jnp.int32),
            pltpu.VMEM((_CH,), jnp.int32),
            pltpu.VMEM((_CH,), jnp.int32),
            pltpu.VMEM((_CH,), jnp.int32),
            pltpu.VMEM((_CH,), jnp.float32),
            pltpu.VMEM((_CH,), jnp.float32),
            pltpu.VMEM((_CH,), jnp.int32),
            pltpu.VMEM((_CH,), jnp.int32),
            pltpu.VMEM((_CH,), jnp.float32),
            pltpu.VMEM((_CH,), jnp.float32),
            pltpu.VMEM((_CH,), jnp.float32),
            pltpu.VMEM((640,), jnp.float32),
            pltpu.VMEM((dh,), jnp.float32),
            pltpu.VMEM_SHARED((nodes, dh), jnp.float32),
            pltpu.VMEM_SHARED((nodes,), jnp.float32),
            pltpu.SemaphoreType.DMA,
            pltpu.SemaphoreType.DMA,
            pltpu.SemaphoreType.DMA,
            pltpu.SemaphoreType.DMA,
            pltpu.SemaphoreType.DMA,
            pltpu.SemaphoreType.DMA,
        ],
    )
    return k(xlo, xhi, ei_flat, ew, b_lin)


def kernel(x, edge_index, edge_weight, W_lin, b_lin, W_u, U_u, b_u,
           W_r, U_r, b_r, W_h, U_h, b_h):
    n, d = x.shape
    e = edge_index.shape[1]

    xlo, xhi = _project(x, W_lin, W_u, U_u, b_u, W_r, U_r, b_r, W_h, U_h, b_h)

    total = e // _CH + -(-n // _CH)
    chunks = -(-total // 32) * 2
    ei_flat = edge_index.reshape(2 * e)

    return _aggregate(xlo, xhi, ei_flat, edge_weight, b_lin, n, chunks, e)
